# baseline (speedup 1.0000x reference)
import numpy as np
import jax
import jax.numpy as jnp
from jax.experimental import pallas as pl
from jax.experimental.pallas import tpu as pltpu

W_MAX = 8
STEP = 1
LEAK = 2
KS = (W_MAX - 1) * (STEP + LEAK)
THETA = 512
FODEP = KS
NEURONS = 512
SYNAPSES = 512
BATCH = 32
TIME = 64
TOUT = TIME + KS + 1
TPOT = 96
NV = W_MAX - 1
BN = BATCH * NEURONS


def _base_table():
    t = np.arange(KS, dtype=np.float64)[None, :]
    w = np.arange(W_MAX, dtype=np.float64)[:, None]
    w_step = np.maximum(np.floor(1.0 + t / STEP), 0.0)
    w_leak = np.maximum(np.ceil(w + ((w - 1.0) * STEP - t) / LEAK), 0.0)
    return np.minimum(w_step, w_leak).astype(np.int64)


def _a_cat():
    base = _base_table()
    A = np.zeros((TPOT, NV * TIME), dtype=np.float32)
    for s in range(NV):
        v = s + 1
        for tp in range(TOUT):
            lo = max(0, tp - 1 - (KS - 1))
            hi = min(TIME - 1, tp - 1)
            for t in range(lo, hi + 1):
                A[tp, s * TIME + t] = float(base[v, tp - 1 - t])
    return A


def _fc_kernel(xt_ref, wt_ref, acat_ref, out_ref,
               u_ref, elig_ref, fires_ref):
    wt = wt_ref[...]
    for s in range(NV):
        m = (wt == (s + 1)).astype(jnp.bfloat16)
        u = jnp.dot(xt_ref[...], m,
                    preferred_element_type=jnp.float32)
        u_ref[pl.ds(s * TIME, TIME), :, :] = u.reshape(TIME, BATCH, NEURONS)

    u_all = u_ref[...].reshape(NV * TIME, BN)
    pot = jnp.dot(acat_ref[...], u_all,
                  preferred_element_type=jnp.float32)
    pot_i = pot.astype(jnp.int32)

    iota_n = jax.lax.broadcasted_iota(jnp.int32, (TPOT, BN), 1) & (NEURONS - 1)
    key = pot_i * 512 + (NEURONS - 1 - iota_n)
    keys = jnp.max(key.reshape(TPOT, BATCH, NEURONS), axis=2)
    win = (NEURONS - 1) - (keys & (NEURONS - 1))
    elig_ref[...] = ((keys >> 9) > THETA).astype(jnp.int32)

    def body(t, dep):
        e = elig_ref[pl.ds(t, 1), :]
        fire = jnp.where((e > 0) & (dep == 0), 1, 0)
        fires_ref[pl.ds(t, 1), :] = fire
        return jnp.maximum(dep + fire * (FODEP + 1) - 1, 0)

    jax.lax.fori_loop(0, TOUT, body, jnp.zeros((1, BATCH), jnp.int32))

    fires = fires_ref[...]
    iota_n3 = jax.lax.broadcasted_iota(jnp.int32, (TPOT, BATCH, NEURONS), 2)
    hit = (win[:, :, None] == iota_n3) & (fires[:, :, None] > 0)
    out_ref[...] = hit.astype(jnp.int32)


def kernel(input_spikes, weight):
    B, C, S, T = input_spikes.shape
    x = input_spikes.reshape(B, C * S, T)
    xt = x.transpose(2, 0, 1).reshape(T * B, C * S).astype(jnp.bfloat16)
    wtT = weight.T.astype(jnp.int32)
    acat = jnp.asarray(_a_cat())

    out3 = pl.pallas_call(
        _fc_kernel,
        out_shape=jax.ShapeDtypeStruct((TPOT, BATCH, NEURONS), jnp.int32),
        scratch_shapes=[
            pltpu.VMEM((NV * TIME, BATCH, NEURONS), jnp.float32),
            pltpu.VMEM((TPOT, BATCH), jnp.int32),
            pltpu.VMEM((TPOT, BATCH), jnp.int32),
        ],
    )(xt, wtT, acat)

    out = out3[:TOUT].transpose(1, 2, 0)
    return out.reshape(B, 1, NEURONS, TOUT)


# speedup vs baseline: 1.1091x; 1.1091x over previous
import numpy as np
import jax
import jax.numpy as jnp
from jax.experimental import pallas as pl
from jax.experimental.pallas import tpu as pltpu

W_MAX = 8
STEP = 1
LEAK = 2
KS = (W_MAX - 1) * (STEP + LEAK)
THETA = 512
FODEP = KS
NEURONS = 512
SYNAPSES = 512
BATCH = 32
TIME = 64
TOUT = TIME + KS + 1
TPOT = 96
NV = W_MAX - 1
BN = BATCH * NEURONS


def _base_table():
    t = np.arange(KS, dtype=np.float64)[None, :]
    w = np.arange(W_MAX, dtype=np.float64)[:, None]
    w_step = np.maximum(np.floor(1.0 + t / STEP), 0.0)
    w_leak = np.maximum(np.ceil(w + ((w - 1.0) * STEP - t) / LEAK), 0.0)
    return np.minimum(w_step, w_leak).astype(np.int64)


def _a_cat():
    base = _base_table()
    A = np.zeros((TPOT, NV * TIME), dtype=np.float32)
    for s in range(NV):
        v = s + 1
        for tp in range(TOUT):
            lo = max(0, tp - 1 - (KS - 1))
            hi = min(TIME - 1, tp - 1)
            for t in range(lo, hi + 1):
                A[tp, s * TIME + t] = float(base[v, tp - 1 - t])
    return A


def _fc_kernel(xt_ref, wt_ref, acat_ref, out_ref,
               u_ref, elig_ref, fires_ref):
    wt = wt_ref[...]
    xt = xt_ref[...].astype(jnp.bfloat16)
    for s in range(NV):
        m = (wt == (s + 1)).astype(jnp.bfloat16)
        u = jnp.dot(xt, m,
                    preferred_element_type=jnp.float32)
        u_ref[pl.ds(s * TIME, TIME), :, :] = u.reshape(TIME, BATCH, NEURONS)

    u_all = u_ref[...].reshape(NV * TIME, BN)
    pot = jnp.dot(acat_ref[...], u_all,
                  preferred_element_type=jnp.float32)
    pot_i = pot.astype(jnp.int32)

    iota_n = jax.lax.broadcasted_iota(jnp.int32, (TPOT, BN), 1) & (NEURONS - 1)
    key = pot_i * 512 + (NEURONS - 1 - iota_n)
    keys = jnp.max(key.reshape(TPOT, BATCH, NEURONS), axis=2)
    win = (NEURONS - 1) - (keys & (NEURONS - 1))
    elig_ref[...] = ((keys >> 9) > THETA).astype(jnp.int32)

    def body(t, dep):
        e = elig_ref[pl.ds(t, 1), :]
        fire = jnp.where((e > 0) & (dep == 0), 1, 0)
        fires_ref[pl.ds(t, 1), :] = fire
        return jnp.maximum(dep + fire * (FODEP + 1) - 1, 0)

    jax.lax.fori_loop(0, TOUT, body, jnp.zeros((1, BATCH), jnp.int32))

    winT = jnp.transpose(win[:TOUT])
    firesT = jnp.transpose(fires_ref[pl.ds(0, TOUT), :])
    iota_n3 = jax.lax.broadcasted_iota(jnp.int32, (BATCH, NEURONS, TOUT), 1)
    hit = (winT[:, None, :] == iota_n3) & (firesT[:, None, :] > 0)
    out_ref[...] = hit.astype(jnp.int32)


def kernel(input_spikes, weight):
    B, C, S, T = input_spikes.shape
    x = input_spikes.reshape(B, C * S, T)
    xt = x.transpose(2, 0, 1).reshape(T * B, C * S).astype(jnp.int8)
    wtT = weight.T.astype(jnp.int32)
    acat = jnp.asarray(_a_cat())

    out3 = pl.pallas_call(
        _fc_kernel,
        out_shape=jax.ShapeDtypeStruct((BATCH, NEURONS, TOUT), jnp.int32),
        scratch_shapes=[
            pltpu.VMEM((NV * TIME, BATCH, NEURONS), jnp.float32),
            pltpu.VMEM((TPOT, BATCH), jnp.int32),
            pltpu.VMEM((TPOT, BATCH), jnp.int32),
        ],
    )(xt, wtT, acat)

    return out3.reshape(B, 1, NEURONS, TOUT)
